# baseline (speedup 1.0000x reference)
import functools

import jax
import jax.numpy as jnp
from jax import lax
from jax.experimental import pallas as pl
from jax.experimental.pallas import tpu as pltpu
from jax.experimental.pallas import tpu_sc as plsc

_BUFF = 10
_B, _N, _H, _W = 16, 512, 512, 512
_NC, _NS, _L = 2, 16, 16
_NW = _NC * _NS
_PT = (_B * _N) // _NW
_STEPS = _PT // _L
_ROWS = _PT * 4
_IMG = _H * _W


def _integral_body(gt_ref, ps_ref, e_ref):
    m = (gt_ref[0, 0] == ps_ref[0]).astype(jnp.float32)
    row = lax.broadcasted_iota(jnp.int32, (_H, _H), 0)
    col = lax.broadcasted_iota(jnp.int32, (_H, _H), 1)
    ls = (row > col).astype(jnp.float32)
    us = (row < col).astype(jnp.float32)
    a = jnp.dot(ls, m, preferred_element_type=jnp.float32)
    e_ref[0] = jnp.dot(a, us, preferred_element_type=jnp.float32)


def _integral_images(gt_segment, ps_segments):
    return pl.pallas_call(
        _integral_body,
        grid=(_B,),
        in_specs=[
            pl.BlockSpec((1, 1, _H, _W), lambda b: (b, 0, 0, 0)),
            pl.BlockSpec((1, _H, _W), lambda b: (b, 0, 0)),
        ],
        out_specs=pl.BlockSpec((1, _H, _W), lambda b: (b, 0, 0)),
        out_shape=jax.ShapeDtypeStruct((_B, _H, _W), jnp.float32),
    )(gt_segment, ps_segments)


def _sc_body(pts_h, tab_h, sgt_h,
             xy_v, ramp_v, idx_v, den_v, vals_v, sgt_v, sem):
    wid = lax.axis_index("s") * _NC + lax.axis_index("c")
    base = wid * _PT
    lane = lax.iota(jnp.int32, _L)

    for i in range(_STEPS):
        pos = base * 2 + i * 2 * _L + 2 * lane
        ramp_v[pl.ds(i * _L, _L)] = pos
        ramp_v[pl.ds(_PT + i * _L, _L)] = pos + 1
    pcopies = [
        pltpu.async_copy(pts_h.at[ramp_v.at[pl.ds(k * 128, 128)]],
                         xy_v.at[pl.ds(k * 128, 128)], sem)
        for k in range(2 * _PT // 128)
    ]
    for c in pcopies:
        c.wait()

    for i in range(_STEPS):
        x = xy_v[pl.ds(i * _L, _L)]
        y = xy_v[pl.ds(_PT + i * _L, _L)]
        px = ((x + 1.0) * (0.5 * _W)).astype(jnp.int32)
        py = ((y + 1.0) * (0.5 * _H)).astype(jnp.int32)
        tx = jnp.clip(px - _BUFF, 0, _W - 1)
        bx = jnp.clip(px + _BUFF, 0, _W - 1)
        ty = jnp.clip(py - _BUFF, 0, _H - 1)
        by = jnp.clip(py + _BUFF, 0, _H - 1)
        den_v[pl.ds(i * _L, _L)] = ((bx - tx) * (by - ty)).astype(jnp.float32)
        g = base + i * _L + lane
        fb = lax.shift_right_logical(g, 9) * _IMG
        yb = by * _W
        yt = ty * _W
        for c, f in enumerate((fb + yb + bx, fb + yt + bx,
                               fb + yb + tx, fb + yt + tx)):
            idx_v[pl.ds(c * _PT + i * _L, _L)] = f

    copies = [
        pltpu.async_copy(tab_h.at[idx_v.at[pl.ds(k * 128, 128)]],
                         vals_v.at[pl.ds(k * 128, 128)], sem)
        for k in range(_ROWS // 128)
    ]
    for c in copies:
        c.wait()

    for i in range(_STEPS):
        v = [vals_v[pl.ds(c * _PT + i * _L, _L)] for c in range(4)]
        cnt = v[0] - v[1] - v[2] + v[3]
        s = jnp.clip(cnt / den_v[pl.ds(i * _L, _L)], 0.0, 1.0)
        sgt_v[pl.ds(i * _L, _L)] = s

    pltpu.sync_copy(sgt_v, sgt_h.at[wid])


def _sc_scores_gt(pts, table):
    mesh = plsc.VectorSubcoreMesh(core_axis_name="c", subcore_axis_name="s")
    run = pl.kernel(
        _sc_body,
        out_type=jax.ShapeDtypeStruct((_NW, _PT), jnp.float32),
        mesh=mesh,
        scratch_types=[
            pltpu.VMEM((_PT * 2,), jnp.float32),
            pltpu.VMEM((_PT * 2,), jnp.int32),
            pltpu.VMEM((_ROWS,), jnp.int32),
            pltpu.VMEM((_PT,), jnp.float32),
            pltpu.VMEM((_ROWS,), jnp.float32),
            pltpu.VMEM((_PT,), jnp.float32),
            pltpu.SemaphoreType.DMA,
        ],
    )
    return run(pts, table)


def _loss_body(s_ref, g_ref, out_ref):
    d = s_ref[...] - g_ref[...]
    out_ref[0, 0] = jnp.sum(d * d) * (1.0 / (_B * _N))


def _loss(scores, scores_gt):
    out = pl.pallas_call(
        _loss_body,
        out_specs=pl.BlockSpec(memory_space=pltpu.SMEM),
        out_shape=jax.ShapeDtypeStruct((1, 1), jnp.float32),
    )(scores, scores_gt)
    return out[0, 0]


def kernel(scores, points, gt_segment, ps_segments):
    e = _integral_images(gt_segment, ps_segments)
    table = e.reshape(_B * _IMG)
    sgt = _sc_scores_gt(points.reshape(-1), table).reshape(_B, _N)
    return (_loss(scores, sgt), sgt)


# speedup vs baseline: 1.2212x; 1.2212x over previous
import functools

import jax
import jax.numpy as jnp
from jax import lax
from jax.experimental import pallas as pl
from jax.experimental.pallas import tpu as pltpu
from jax.experimental.pallas import tpu_sc as plsc

_BUFF = 10
_B, _N, _H, _W = 16, 512, 512, 512
_NC, _NS, _L = 2, 16, 16
_NW = _NC * _NS
_PT = (_B * _N) // _NW
_STEPS = _PT // _L
_ROWS = _PT * 4
_IMG = _H * _W


def _integral_body(gt_ref, ps_ref, e_ref):
    m = (gt_ref[0, 0] == ps_ref[0]).astype(jnp.float32)
    row = lax.broadcasted_iota(jnp.int32, (_H, _H), 0)
    col = lax.broadcasted_iota(jnp.int32, (_H, _H), 1)
    ls = (row > col).astype(jnp.float32)
    us = (row < col).astype(jnp.float32)
    a = jnp.dot(ls, m, preferred_element_type=jnp.float32)
    e = jnp.dot(a, us, preferred_element_type=jnp.float32)
    e_ref[...] = e.reshape(_IMG)


def _integral_images(gt_segment, ps_segments):
    return pl.pallas_call(
        _integral_body,
        grid=(_B,),
        in_specs=[
            pl.BlockSpec((1, 1, _H, _W), lambda b: (b, 0, 0, 0)),
            pl.BlockSpec((1, _H, _W), lambda b: (b, 0, 0)),
        ],
        out_specs=pl.BlockSpec((_IMG,), lambda b: (b,)),
        out_shape=jax.ShapeDtypeStruct((_B * _IMG,), jnp.float32),
    )(gt_segment, ps_segments)


def _sc_body(pts_h, tab_h, sgt_h,
             xy_v, ramp_v, idx_v, den_v, vals_v, sgt_v, sem):
    wid = lax.axis_index("s") * _NC + lax.axis_index("c")
    base = wid * _PT
    lane = lax.iota(jnp.int32, _L)

    for i in range(_STEPS):
        pos = base * 2 + i * 2 * _L + 2 * lane
        ramp_v[pl.ds(i * _L, _L)] = pos
        ramp_v[pl.ds(_PT + i * _L, _L)] = pos + 1
    pcopies = [
        pltpu.async_copy(pts_h.at[ramp_v.at[pl.ds(k * 128, 128)]],
                         xy_v.at[pl.ds(k * 128, 128)], sem)
        for k in range(2 * _PT // 128)
    ]
    for c in pcopies:
        c.wait()

    for i in range(_STEPS):
        x = xy_v[pl.ds(i * _L, _L)]
        y = xy_v[pl.ds(_PT + i * _L, _L)]
        px = ((x + 1.0) * (0.5 * _W)).astype(jnp.int32)
        py = ((y + 1.0) * (0.5 * _H)).astype(jnp.int32)
        tx = jnp.clip(px - _BUFF, 0, _W - 1)
        bx = jnp.clip(px + _BUFF, 0, _W - 1)
        ty = jnp.clip(py - _BUFF, 0, _H - 1)
        by = jnp.clip(py + _BUFF, 0, _H - 1)
        den_v[pl.ds(i * _L, _L)] = ((bx - tx) * (by - ty)).astype(jnp.float32)
        g = base + i * _L + lane
        fb = lax.shift_right_logical(g, 9) * _IMG
        yb = by * _W
        yt = ty * _W
        for c, f in enumerate((fb + yb + bx, fb + yt + bx,
                               fb + yb + tx, fb + yt + tx)):
            idx_v[pl.ds(c * _PT + i * _L, _L)] = f

    copies = [
        pltpu.async_copy(tab_h.at[idx_v.at[pl.ds(k * 128, 128)]],
                         vals_v.at[pl.ds(k * 128, 128)], sem)
        for k in range(_ROWS // 128)
    ]
    for c in copies:
        c.wait()

    for i in range(_STEPS):
        v = [vals_v[pl.ds(c * _PT + i * _L, _L)] for c in range(4)]
        cnt = v[0] - v[1] - v[2] + v[3]
        s = jnp.clip(cnt / den_v[pl.ds(i * _L, _L)], 0.0, 1.0)
        sgt_v[pl.ds(i * _L, _L)] = s

    pltpu.sync_copy(sgt_v, sgt_h.at[wid])


def _sc_scores_gt(pts, table):
    mesh = plsc.VectorSubcoreMesh(core_axis_name="c", subcore_axis_name="s")
    run = pl.kernel(
        _sc_body,
        out_type=jax.ShapeDtypeStruct((_NW, _PT), jnp.float32),
        mesh=mesh,
        scratch_types=[
            pltpu.VMEM((_PT * 2,), jnp.float32),
            pltpu.VMEM((_PT * 2,), jnp.int32),
            pltpu.VMEM((_ROWS,), jnp.int32),
            pltpu.VMEM((_PT,), jnp.float32),
            pltpu.VMEM((_ROWS,), jnp.float32),
            pltpu.VMEM((_PT,), jnp.float32),
            pltpu.SemaphoreType.DMA,
        ],
    )
    return run(pts, table)


def _loss_body(s_ref, g_ref, out_ref):
    d = s_ref[...] - g_ref[...]
    out_ref[0, 0] = jnp.sum(d * d) * (1.0 / (_B * _N))


def _loss(scores, scores_gt):
    out = pl.pallas_call(
        _loss_body,
        out_specs=pl.BlockSpec(memory_space=pltpu.SMEM),
        out_shape=jax.ShapeDtypeStruct((1, 1), jnp.float32),
    )(scores, scores_gt)
    return out[0, 0]


def kernel(scores, points, gt_segment, ps_segments):
    table = _integral_images(gt_segment, ps_segments)
    sgt = _sc_scores_gt(points.reshape(-1), table).reshape(_B, _N)
    return (_loss(scores, sgt), sgt)


# speedup vs baseline: 1.2706x; 1.0404x over previous
import functools

import jax
import jax.numpy as jnp
from jax import lax
from jax.experimental import pallas as pl
from jax.experimental.pallas import tpu as pltpu
from jax.experimental.pallas import tpu_sc as plsc

_BUFF = 10
_B, _N, _H, _W = 16, 512, 512, 512
_NC, _NS, _L = 2, 16, 16
_NW = _NC * _NS
_PT = (_B * _N) // _NW
_STEPS = _PT // _L
_ROWS = _PT * 4
_IMG = _H * _W


def _integral_body(gt_ref, ps_ref, pts_ref, e_ref, pxy_ref):
    m = (gt_ref[0, 0] == ps_ref[0]).astype(jnp.float32)
    row = lax.broadcasted_iota(jnp.int32, (_H, _H), 0)
    col = lax.broadcasted_iota(jnp.int32, (_H, _H), 1)
    ls = (row > col).astype(jnp.float32)
    us = (row < col).astype(jnp.float32)
    a = jnp.dot(ls, m, preferred_element_type=jnp.float32)
    e = jnp.dot(a, us, preferred_element_type=jnp.float32)
    e_ref[...] = e.reshape(_IMG)
    eye = (row == col).astype(jnp.float32)
    pts_t = lax.dot_general(pts_ref[0], eye, (((0,), (0,)), ((), ())),
                            preferred_element_type=jnp.float32)
    pxy_ref[...] = pts_t.reshape(2 * _N)


def _integral_images(gt_segment, ps_segments, points):
    return pl.pallas_call(
        _integral_body,
        grid=(_B,),
        in_specs=[
            pl.BlockSpec((1, 1, _H, _W), lambda b: (b, 0, 0, 0)),
            pl.BlockSpec((1, _H, _W), lambda b: (b, 0, 0)),
            pl.BlockSpec((1, _N, 2), lambda b: (b, 0, 0)),
        ],
        out_specs=[
            pl.BlockSpec((_IMG,), lambda b: (b,)),
            pl.BlockSpec((2 * _N,), lambda b: (b,)),
        ],
        out_shape=[
            jax.ShapeDtypeStruct((_B * _IMG,), jnp.float32),
            jax.ShapeDtypeStruct((_B * 2 * _N,), jnp.float32),
        ],
    )(gt_segment, ps_segments, points)


def _sc_body(pxy_h, tab_h, sgt_h,
             xy_v, idx_v, den_v, vals_v, sgt_v, sem):
    wid = lax.axis_index("s") * _NC + lax.axis_index("c")
    base = wid * _PT

    b = lax.shift_right_logical(wid, 1)
    xoff = b * (2 * _N) + jnp.bitwise_and(wid, 1) * _PT
    pltpu.sync_copy(pxy_h.at[pl.ds(xoff, _PT)], xy_v.at[pl.ds(0, _PT)])
    pltpu.sync_copy(pxy_h.at[pl.ds(xoff + _N, _PT)], xy_v.at[pl.ds(_PT, _PT)])

    for i in range(_STEPS):
        x = xy_v[pl.ds(i * _L, _L)]
        y = xy_v[pl.ds(_PT + i * _L, _L)]
        px = ((x + 1.0) * (0.5 * _W)).astype(jnp.int32)
        py = ((y + 1.0) * (0.5 * _H)).astype(jnp.int32)
        tx = jnp.clip(px - _BUFF, 0, _W - 1)
        bx = jnp.clip(px + _BUFF, 0, _W - 1)
        ty = jnp.clip(py - _BUFF, 0, _H - 1)
        by = jnp.clip(py + _BUFF, 0, _H - 1)
        den_v[pl.ds(i * _L, _L)] = ((bx - tx) * (by - ty)).astype(jnp.float32)
        fb = b * _IMG
        yb = by * _W
        yt = ty * _W
        for c, f in enumerate((fb + yb + bx, fb + yt + bx,
                               fb + yb + tx, fb + yt + tx)):
            idx_v[pl.ds(c * _PT + i * _L, _L)] = f

    copies = [
        pltpu.async_copy(tab_h.at[idx_v.at[pl.ds(k * 128, 128)]],
                         vals_v.at[pl.ds(k * 128, 128)], sem)
        for k in range(_ROWS // 128)
    ]
    for c in copies:
        c.wait()

    for i in range(_STEPS):
        v = [vals_v[pl.ds(c * _PT + i * _L, _L)] for c in range(4)]
        cnt = v[0] - v[1] - v[2] + v[3]
        s = jnp.clip(cnt / den_v[pl.ds(i * _L, _L)], 0.0, 1.0)
        sgt_v[pl.ds(i * _L, _L)] = s

    pltpu.sync_copy(sgt_v, sgt_h.at[wid])


def _sc_scores_gt(pxy, table):
    mesh = plsc.VectorSubcoreMesh(core_axis_name="c", subcore_axis_name="s")
    run = pl.kernel(
        _sc_body,
        out_type=jax.ShapeDtypeStruct((_NW, _PT), jnp.float32),
        mesh=mesh,
        scratch_types=[
            pltpu.VMEM((_PT * 2,), jnp.float32),
            pltpu.VMEM((_ROWS,), jnp.int32),
            pltpu.VMEM((_PT,), jnp.float32),
            pltpu.VMEM((_ROWS,), jnp.float32),
            pltpu.VMEM((_PT,), jnp.float32),
            pltpu.SemaphoreType.DMA,
        ],
    )
    return run(pxy, table)


def _loss_body(s_ref, g_ref, out_ref):
    s = s_ref[...]
    g = g_ref[...].reshape(_B, 2, _N // 2)
    d0 = g[:, 0, :] - s[:, : _N // 2]
    d1 = g[:, 1, :] - s[:, _N // 2:]
    out_ref[0, 0] = (jnp.sum(d0 * d0) + jnp.sum(d1 * d1)) * (1.0 / (_B * _N))


def _loss(scores, scores_gt):
    out = pl.pallas_call(
        _loss_body,
        out_specs=pl.BlockSpec(memory_space=pltpu.SMEM),
        out_shape=jax.ShapeDtypeStruct((1, 1), jnp.float32),
    )(scores, scores_gt)
    return out[0, 0]


def kernel(scores, points, gt_segment, ps_segments):
    table, pxy = _integral_images(gt_segment, ps_segments, points)
    sgt32 = _sc_scores_gt(pxy, table)
    return (_loss(scores, sgt32), sgt32.reshape(_B, _N))
